# baseline (device time: 87204 ns/iter reference)
import os

import jax
import jax.numpy as jnp
from jax import lax
from jax.experimental import pallas as pl
from jax.experimental.pallas import tpu as pltpu

ABLATE = int(os.environ.get("ABLATE", "0"))
DO_RS = ABLATE < 2
DO_AG = ABLATE < 1

N_DEV = 16
B, S, C_IN, C_OUT = 4, 1024, 512, 512
ROWS = B * S
HALF = ROWS // 2
N8 = 8
HOPS8 = N8 - 1
CHUNK8 = HALF // N8
CCHUNK = 128
CPB = S // CCHUNK

RING_A = [0, 4, 8, 12, 15, 11, 7, 3]
RING_B = [1, 5, 9, 13, 14, 10, 6, 2]
Q = [0] * N_DEV
RIGHT8 = [0] * N_DEV
LEFT8 = [0] * N_DEV
PARTNER = [0] * N_DEV
for _i in range(N8):
    _a, _b = RING_A[_i], RING_B[_i]
    Q[_a] = Q[_b] = _i
    RIGHT8[_a], LEFT8[_a] = RING_A[(_i + 1) % N8], RING_A[(_i - 1) % N8]
    RIGHT8[_b], LEFT8[_b] = RING_B[(_i + 1) % N8], RING_B[(_i - 1) % N8]
    PARTNER[_a], PARTNER[_b] = _b, _a


def _lut(table, idx):
    acc = jnp.int32(table[0])
    for i in range(1, len(table)):
        acc = jnp.where(idx == i, jnp.int32(table[i]), acc)
    return acc


def kernel(x, k, Wp):
    def body(x_ref, k_ref, w_ref, out_ref,
             cw_sb, ccw_sb, cw_stage, ccw_stage,
             cw_exsb, ccw_exsb, cw_exst, ccw_exst,
             cw_ag, ccw_ag,
             cw_rs_send, cw_rs_recv, cw_ag_send, cw_ag_recv,
             ccw_rs_send, ccw_rs_recv, ccw_ag_send, ccw_ag_recv,
             ex_send, ex_recv):
        my = lax.axis_index("i")
        q = _lut(Q, my)
        right = _lut(RIGHT8, my)
        left = _lut(LEFT8, my)
        partner = _lut(PARTNER, my)

        barrier = pltpu.get_barrier_semaphore()
        for nbr in (left, right, partner):
            pl.semaphore_signal(barrier, inc=1, device_id=(nbr,),
                                device_id_type=pl.DeviceIdType.MESH)
        pl.semaphore_wait(barrier, 3)

        kv = k_ref[:, :]
        wv_bf = w_ref[:, :].astype(jnp.bfloat16)

        def compute_chunk(c, b_base, half_base):
            b = lax.div(c, CPB) + b_base
            s0 = pl.multiple_of(lax.rem(c, CPB) * CCHUNK, CCHUNK)
            xc = x_ref[b, pl.ds(s0, CCHUNK), :]
            hs = pl.multiple_of(jnp.maximum(s0 - 8, 0), 8)
            halo = x_ref[b, pl.ds(hs, 8), :][5:8]
            halo = jnp.where(s0 == 0, jnp.zeros_like(halo), halo)
            xe = jnp.concatenate([halo, xc], axis=0)
            accv = xe[3:3 + CCHUNK] * kv[3][None, :]
            for t in range(3):
                accv = accv + xe[t:t + CCHUNK] * kv[t][None, :]
            av = accv / (1.0 + jnp.exp(-accv))
            out_ref[pl.ds(half_base + c * CCHUNK, CCHUNK), :] = (
                jax.lax.dot_general(
                    av.astype(jnp.bfloat16), wv_bf, (((1,), (0,)), ((), ())),
                    preferred_element_type=jnp.float32,
                )
            )

        def compute_c8(c8, b_base, half_base):
            compute_chunk(2 * c8, b_base, half_base)
            compute_chunk(2 * c8 + 1, b_base, half_base)

        def chunk8(i):
            return lax.rem(q + i + 2 * N8, N8)

        def rd(src, dst, send_sem, recv_sem, dev):
            return pltpu.make_async_remote_copy(
                src_ref=src, dst_ref=dst, send_sem=send_sem,
                recv_sem=recv_sem, device_id=(dev,),
                device_id_type=pl.DeviceIdType.MESH,
            )

        def out_chunk(off):
            return out_ref[pl.ds(off, CHUNK8), :]

        compute_c8(chunk8(0), 0, 0)
        compute_c8(chunk8(0), 2, HALF)
        compute_c8(chunk8(-1), 0, 0)
        compute_c8(chunk8(1), 2, HALF)

        pending_sends = []

        cw_sb[0, :, :] = out_chunk(chunk8(0) * CHUNK8).astype(jnp.bfloat16)
        ccw_sb[0, :, :] = out_chunk(
            HALF + chunk8(0) * CHUNK8).astype(jnp.bfloat16)

        for s in range(HOPS8):
            if DO_RS:
                cw = rd(cw_sb.at[s], cw_stage.at[s],
                        cw_rs_send.at[s], cw_rs_recv.at[s], right)
                ccw = rd(ccw_sb.at[s], ccw_stage.at[s],
                         ccw_rs_send.at[s], ccw_rs_recv.at[s], left)
                cw.start()
                ccw.start()
                pending_sends += [cw, ccw]
            if s < HOPS8 - 1:
                compute_c8(chunk8(-s - 2), 0, 0)
                compute_c8(chunk8(s + 2), 2, HALF)
            if not DO_RS:
                continue
            cw.wait_recv()
            off = chunk8(-s - 1) * CHUNK8
            summed = out_chunk(off) + cw_stage[s, :, :].astype(jnp.float32)
            out_ref[pl.ds(off, CHUNK8), :] = summed
            if s < HOPS8 - 1:
                cw_sb[s + 1, :, :] = summed.astype(jnp.bfloat16)
            else:
                cw_exsb[:, :] = summed.astype(jnp.bfloat16)
            ccw.wait_recv()
            off = HALF + chunk8(s + 1) * CHUNK8
            summed = out_chunk(off) + ccw_stage[s, :, :].astype(jnp.float32)
            out_ref[pl.ds(off, CHUNK8), :] = summed
            if s < HOPS8 - 1:
                ccw_sb[s + 1, :, :] = summed.astype(jnp.bfloat16)
            else:
                ccw_exsb[:, :] = summed.astype(jnp.bfloat16)

        if DO_RS:
            cwx = rd(cw_exsb, cw_exst, ex_send.at[0], ex_recv.at[0], partner)
            ccwx = rd(ccw_exsb, ccw_exst, ex_send.at[1], ex_recv.at[1],
                      partner)
            cwx.start()
            ccwx.start()
            pending_sends += [cwx, ccwx]
            cwx.wait_recv()
            off = chunk8(1) * CHUNK8
            final = out_chunk(off) + cw_exst[:, :].astype(jnp.float32)
            out_ref[pl.ds(off, CHUNK8), :] = final
            cw_ag[pl.ds(off, CHUNK8), :] = final.astype(jnp.bfloat16)
            ccwx.wait_recv()
            off = chunk8(-1) * CHUNK8
            final = out_chunk(HALF + off) + ccw_exst[:, :].astype(jnp.float32)
            out_ref[pl.ds(HALF + off, CHUNK8), :] = final
            ccw_ag[pl.ds(off, CHUNK8), :] = final.astype(jnp.bfloat16)

        for s in range(HOPS8 if DO_AG else 0):
            cw_off = chunk8(1 - s) * CHUNK8
            ccw_off = chunk8(s - 1) * CHUNK8
            cw = rd(cw_ag.at[pl.ds(cw_off, CHUNK8), :],
                    cw_ag.at[pl.ds(cw_off, CHUNK8), :],
                    cw_ag_send.at[s], cw_ag_recv.at[s], right)
            ccw = rd(ccw_ag.at[pl.ds(ccw_off, CHUNK8), :],
                     ccw_ag.at[pl.ds(ccw_off, CHUNK8), :],
                     ccw_ag_send.at[s], ccw_ag_recv.at[s], left)
            cw.start()
            ccw.start()
            pending_sends += [cw, ccw]
            if s > 0:
                o = chunk8(1 - s) * CHUNK8
                out_ref[pl.ds(o, CHUNK8), :] = (
                    cw_ag[pl.ds(o, CHUNK8), :].astype(jnp.float32))
                o = chunk8(s - 1) * CHUNK8
                out_ref[pl.ds(HALF + o, CHUNK8), :] = (
                    ccw_ag[pl.ds(o, CHUNK8), :].astype(jnp.float32))
            cw.wait_recv()
            ccw.wait_recv()
        if DO_AG:
            o = chunk8(-HOPS8 + 1) * CHUNK8
            out_ref[pl.ds(o, CHUNK8), :] = (
                cw_ag[pl.ds(o, CHUNK8), :].astype(jnp.float32))
            o = chunk8(HOPS8 - 1) * CHUNK8
            out_ref[pl.ds(HALF + o, CHUNK8), :] = (
                ccw_ag[pl.ds(o, CHUNK8), :].astype(jnp.float32))

        for r in pending_sends:
            r.wait_send()

    out = pl.pallas_call(
        body,
        out_shape=jax.ShapeDtypeStruct((ROWS, C_OUT), jnp.float32),
        in_specs=[pl.BlockSpec(memory_space=pltpu.VMEM)] * 3,
        out_specs=pl.BlockSpec(memory_space=pltpu.VMEM),
        scratch_shapes=[
            pltpu.VMEM((HOPS8, CHUNK8, C_OUT), jnp.bfloat16),
            pltpu.VMEM((HOPS8, CHUNK8, C_OUT), jnp.bfloat16),
            pltpu.VMEM((HOPS8, CHUNK8, C_OUT), jnp.bfloat16),
            pltpu.VMEM((HOPS8, CHUNK8, C_OUT), jnp.bfloat16),
            pltpu.VMEM((CHUNK8, C_OUT), jnp.bfloat16),
            pltpu.VMEM((CHUNK8, C_OUT), jnp.bfloat16),
            pltpu.VMEM((CHUNK8, C_OUT), jnp.bfloat16),
            pltpu.VMEM((CHUNK8, C_OUT), jnp.bfloat16),
            pltpu.VMEM((HALF, C_OUT), jnp.bfloat16),
            pltpu.VMEM((HALF, C_OUT), jnp.bfloat16),
        ] + [pltpu.SemaphoreType.DMA((HOPS8,))] * 8
          + [pltpu.SemaphoreType.DMA((2,))] * 2,
        compiler_params=pltpu.CompilerParams(collective_id=0),
    )(x, k, Wp)
    return out.reshape(B, S, C_OUT)


# device time: 86156 ns/iter; 1.0122x vs baseline; 1.0122x over previous
import os

import jax
import jax.numpy as jnp
from jax import lax
from jax.experimental import pallas as pl
from jax.experimental.pallas import tpu as pltpu

ABLATE = int(os.environ.get("ABLATE", "0"))
DO_RS = ABLATE < 2
DO_AG = ABLATE < 1

N_DEV = 16
B, S, C_IN, C_OUT = 4, 1024, 512, 512
ROWS = B * S
HALF = ROWS // 2
N8 = 8
HOPS8 = N8 - 1
CHUNK8 = HALF // N8
CCHUNK = 128
CPB = S // CCHUNK

RING_A = [0, 4, 8, 12, 15, 11, 7, 3]
RING_B = [1, 5, 9, 13, 14, 10, 6, 2]
Q = [0] * N_DEV
RIGHT8 = [0] * N_DEV
LEFT8 = [0] * N_DEV
PARTNER = [0] * N_DEV
for _i in range(N8):
    _a, _b = RING_A[_i], RING_B[_i]
    Q[_a] = Q[_b] = _i
    RIGHT8[_a], LEFT8[_a] = RING_A[(_i + 1) % N8], RING_A[(_i - 1) % N8]
    RIGHT8[_b], LEFT8[_b] = RING_B[(_i + 1) % N8], RING_B[(_i - 1) % N8]
    PARTNER[_a], PARTNER[_b] = _b, _a


def _lut(table, idx):
    acc = jnp.int32(table[0])
    for i in range(1, len(table)):
        acc = jnp.where(idx == i, jnp.int32(table[i]), acc)
    return acc


def kernel(x, k, Wp):
    def body(x_ref, k_ref, w_ref, out_ref,
             cw_sb, ccw_sb, cw_stage, ccw_stage,
             cw_exsb, ccw_exsb, cw_exst, ccw_exst,
             cw_ag, ccw_ag,
             cw_rs_send, cw_rs_recv, cw_ag_send, cw_ag_recv,
             ccw_rs_send, ccw_rs_recv, ccw_ag_send, ccw_ag_recv,
             ex_send, ex_recv):
        my = lax.axis_index("i")
        q = _lut(Q, my)
        right = _lut(RIGHT8, my)
        left = _lut(LEFT8, my)
        partner = _lut(PARTNER, my)

        barrier = pltpu.get_barrier_semaphore()
        for nbr in (left, right, partner):
            pl.semaphore_signal(barrier, inc=1, device_id=(nbr,),
                                device_id_type=pl.DeviceIdType.MESH)
        pl.semaphore_wait(barrier, 3)

        kv = k_ref[:, :]
        wv_bf = w_ref[:, :].astype(jnp.bfloat16)

        def compute_chunk(c, b_base, half_base):
            b = lax.div(c, CPB) + b_base
            s0 = pl.multiple_of(lax.rem(c, CPB) * CCHUNK, CCHUNK)
            xc = x_ref[b, pl.ds(s0, CCHUNK), :]
            hs = pl.multiple_of(jnp.maximum(s0 - 8, 0), 8)
            halo = x_ref[b, pl.ds(hs, 8), :][5:8]
            halo = jnp.where(s0 == 0, jnp.zeros_like(halo), halo)
            xe = jnp.concatenate([halo, xc], axis=0)
            accv = xe[3:3 + CCHUNK] * kv[3][None, :]
            for t in range(3):
                accv = accv + xe[t:t + CCHUNK] * kv[t][None, :]
            av = accv / (1.0 + jnp.exp(-accv))
            out_ref[pl.ds(half_base + c * CCHUNK, CCHUNK), :] = (
                jax.lax.dot_general(
                    av.astype(jnp.bfloat16), wv_bf, (((1,), (0,)), ((), ())),
                    preferred_element_type=jnp.float32,
                )
            )

        def compute_c8(c8, b_base, half_base):
            compute_chunk(2 * c8, b_base, half_base)
            compute_chunk(2 * c8 + 1, b_base, half_base)

        def chunk8(i):
            return lax.rem(q + i + 2 * N8, N8)

        def rd(src, dst, send_sem, recv_sem, dev):
            return pltpu.make_async_remote_copy(
                src_ref=src, dst_ref=dst, send_sem=send_sem,
                recv_sem=recv_sem, device_id=(dev,),
                device_id_type=pl.DeviceIdType.MESH,
            )

        def out_chunk(off):
            return out_ref[pl.ds(off, CHUNK8), :]

        compute_c8(chunk8(0), 0, 0)
        compute_c8(chunk8(0), 2, HALF)

        pending_sends = []

        cw_sb[0, :, :] = out_chunk(chunk8(0) * CHUNK8).astype(jnp.bfloat16)
        ccw_sb[0, :, :] = out_chunk(
            HALF + chunk8(0) * CHUNK8).astype(jnp.bfloat16)

        for s in range(HOPS8):
            if DO_RS:
                cw = rd(cw_sb.at[s], cw_stage.at[s],
                        cw_rs_send.at[s], cw_rs_recv.at[s], right)
                ccw = rd(ccw_sb.at[s], ccw_stage.at[s],
                         ccw_rs_send.at[s], ccw_rs_recv.at[s], left)
                cw.start()
                ccw.start()
                pending_sends += [cw, ccw]
            compute_c8(chunk8(-s - 1), 0, 0)
            compute_c8(chunk8(s + 1), 2, HALF)
            if not DO_RS:
                continue
            cw.wait_recv()
            off = chunk8(-s - 1) * CHUNK8
            summed = out_chunk(off) + cw_stage[s, :, :].astype(jnp.float32)
            out_ref[pl.ds(off, CHUNK8), :] = summed
            if s < HOPS8 - 1:
                cw_sb[s + 1, :, :] = summed.astype(jnp.bfloat16)
            else:
                cw_exsb[:, :] = summed.astype(jnp.bfloat16)
                cwx = rd(cw_exsb, cw_exst, ex_send.at[0], ex_recv.at[0],
                         partner)
                cwx.start()
                pending_sends.append(cwx)
            ccw.wait_recv()
            off = HALF + chunk8(s + 1) * CHUNK8
            summed = out_chunk(off) + ccw_stage[s, :, :].astype(jnp.float32)
            out_ref[pl.ds(off, CHUNK8), :] = summed
            if s < HOPS8 - 1:
                ccw_sb[s + 1, :, :] = summed.astype(jnp.bfloat16)
            else:
                ccw_exsb[:, :] = summed.astype(jnp.bfloat16)
                ccwx = rd(ccw_exsb, ccw_exst, ex_send.at[1], ex_recv.at[1],
                          partner)
                ccwx.start()
                pending_sends.append(ccwx)

        def ag_rd(dir_ag, off, s, dev, send_sems, recv_sems):
            return rd(dir_ag.at[pl.ds(off, CHUNK8), :],
                      dir_ag.at[pl.ds(off, CHUNK8), :],
                      send_sems.at[s], recv_sems.at[s], dev)

        if DO_RS:
            cwx.wait_recv()
            off = chunk8(1) * CHUNK8
            final = out_chunk(off) + cw_exst[:, :].astype(jnp.float32)
            out_ref[pl.ds(off, CHUNK8), :] = final
            cw_ag[pl.ds(off, CHUNK8), :] = final.astype(jnp.bfloat16)
            if DO_AG:
                prev_cw = ag_rd(cw_ag, off, 0, right, cw_ag_send, cw_ag_recv)
                prev_cw.start()
                pending_sends.append(prev_cw)
            ccwx.wait_recv()
            off = chunk8(-1) * CHUNK8
            final = out_chunk(HALF + off) + ccw_exst[:, :].astype(jnp.float32)
            out_ref[pl.ds(HALF + off, CHUNK8), :] = final
            ccw_ag[pl.ds(off, CHUNK8), :] = final.astype(jnp.bfloat16)
            if DO_AG:
                prev_ccw = ag_rd(ccw_ag, off, 0, left, ccw_ag_send,
                                 ccw_ag_recv)
                prev_ccw.start()
                pending_sends.append(prev_ccw)

        for s in range(1, HOPS8 if DO_AG else 1):
            prev_cw.wait_recv()
            prev_ccw.wait_recv()
            cw_off = chunk8(1 - s) * CHUNK8
            ccw_off = chunk8(s - 1) * CHUNK8
            prev_cw = ag_rd(cw_ag, cw_off, s, right, cw_ag_send, cw_ag_recv)
            prev_ccw = ag_rd(ccw_ag, ccw_off, s, left, ccw_ag_send,
                             ccw_ag_recv)
            prev_cw.start()
            prev_ccw.start()
            pending_sends += [prev_cw, prev_ccw]
            out_ref[pl.ds(cw_off, CHUNK8), :] = (
                cw_ag[pl.ds(cw_off, CHUNK8), :].astype(jnp.float32))
            out_ref[pl.ds(HALF + ccw_off, CHUNK8), :] = (
                ccw_ag[pl.ds(ccw_off, CHUNK8), :].astype(jnp.float32))
        if DO_AG:
            prev_cw.wait_recv()
            prev_ccw.wait_recv()
            o = chunk8(-HOPS8 + 1) * CHUNK8
            out_ref[pl.ds(o, CHUNK8), :] = (
                cw_ag[pl.ds(o, CHUNK8), :].astype(jnp.float32))
            o = chunk8(HOPS8 - 1) * CHUNK8
            out_ref[pl.ds(HALF + o, CHUNK8), :] = (
                ccw_ag[pl.ds(o, CHUNK8), :].astype(jnp.float32))

        for r in pending_sends:
            r.wait_send()

    out = pl.pallas_call(
        body,
        out_shape=jax.ShapeDtypeStruct((ROWS, C_OUT), jnp.float32),
        in_specs=[pl.BlockSpec(memory_space=pltpu.VMEM)] * 3,
        out_specs=pl.BlockSpec(memory_space=pltpu.VMEM),
        scratch_shapes=[
            pltpu.VMEM((HOPS8, CHUNK8, C_OUT), jnp.bfloat16),
            pltpu.VMEM((HOPS8, CHUNK8, C_OUT), jnp.bfloat16),
            pltpu.VMEM((HOPS8, CHUNK8, C_OUT), jnp.bfloat16),
            pltpu.VMEM((HOPS8, CHUNK8, C_OUT), jnp.bfloat16),
            pltpu.VMEM((CHUNK8, C_OUT), jnp.bfloat16),
            pltpu.VMEM((CHUNK8, C_OUT), jnp.bfloat16),
            pltpu.VMEM((CHUNK8, C_OUT), jnp.bfloat16),
            pltpu.VMEM((CHUNK8, C_OUT), jnp.bfloat16),
            pltpu.VMEM((HALF, C_OUT), jnp.bfloat16),
            pltpu.VMEM((HALF, C_OUT), jnp.bfloat16),
        ] + [pltpu.SemaphoreType.DMA((HOPS8,))] * 8
          + [pltpu.SemaphoreType.DMA((2,))] * 2,
        compiler_params=pltpu.CompilerParams(collective_id=0),
    )(x, k, Wp)
    return out.reshape(B, S, C_OUT)


# device time: 76776 ns/iter; 1.1358x vs baseline; 1.1222x over previous
import os

import jax
import jax.numpy as jnp
from jax import lax
from jax.experimental import pallas as pl
from jax.experimental.pallas import tpu as pltpu

ABLATE = int(os.environ.get("ABLATE", "0"))
DO_RS = ABLATE < 2
DO_AG = ABLATE < 1

N_DEV = 16
B, S, C_IN, C_OUT = 4, 1024, 512, 512
ROWS = B * S
HALF = ROWS // 2
N8 = 8
HOPS8 = N8 - 1
CHUNK8 = HALF // N8
SUB = CHUNK8 // 2
CCHUNK = 128
CPB = S // CCHUNK

RING_A = [0, 4, 8, 12, 15, 11, 7, 3]
RING_B = [1, 5, 9, 13, 14, 10, 6, 2]
Q = [0] * N_DEV
RIGHT8 = [0] * N_DEV
LEFT8 = [0] * N_DEV
PARTNER = [0] * N_DEV
for _i in range(N8):
    _a, _b = RING_A[_i], RING_B[_i]
    Q[_a] = Q[_b] = _i
    RIGHT8[_a], LEFT8[_a] = RING_A[(_i + 1) % N8], RING_A[(_i - 1) % N8]
    RIGHT8[_b], LEFT8[_b] = RING_B[(_i + 1) % N8], RING_B[(_i - 1) % N8]
    PARTNER[_a], PARTNER[_b] = _b, _a


def _lut(table, idx):
    acc = jnp.int32(table[0])
    for i in range(1, len(table)):
        acc = jnp.where(idx == i, jnp.int32(table[i]), acc)
    return acc


def kernel(x, k, Wp):
    def body(x_ref, k_ref, w_ref, out_ref,
             cw_sb, ccw_sb, cw_stage, ccw_stage,
             cw_exsb, ccw_exsb, cw_exst, ccw_exst,
             cw_ag, ccw_ag,
             cw_rs_send, cw_rs_recv, ccw_rs_send, ccw_rs_recv,
             cw1_ag_send, cw1_ag_recv, cw2_ag_send, cw2_ag_recv,
             ccw1_ag_send, ccw1_ag_recv, ccw2_ag_send, ccw2_ag_recv,
             ex_send, ex_recv):
        my = lax.axis_index("i")
        q = _lut(Q, my)
        right = _lut(RIGHT8, my)
        left = _lut(LEFT8, my)
        partner = _lut(PARTNER, my)

        barrier = pltpu.get_barrier_semaphore()
        for nbr in (left, right, partner):
            pl.semaphore_signal(barrier, inc=1, device_id=(nbr,),
                                device_id_type=pl.DeviceIdType.MESH)
        pl.semaphore_wait(barrier, 3)

        kv = k_ref[:, :]
        wv_bf = w_ref[:, :].astype(jnp.bfloat16)

        def compute_chunk(c, b_base, half_base):
            b = lax.div(c, CPB) + b_base
            s0 = pl.multiple_of(lax.rem(c, CPB) * CCHUNK, CCHUNK)
            xc = x_ref[b, pl.ds(s0, CCHUNK), :]
            hs = pl.multiple_of(jnp.maximum(s0 - 8, 0), 8)
            halo = x_ref[b, pl.ds(hs, 8), :][5:8]
            halo = jnp.where(s0 == 0, jnp.zeros_like(halo), halo)
            xe = jnp.concatenate([halo, xc], axis=0)
            accv = xe[3:3 + CCHUNK] * kv[3][None, :]
            for t in range(3):
                accv = accv + xe[t:t + CCHUNK] * kv[t][None, :]
            av = accv / (1.0 + jnp.exp(-accv))
            out_ref[pl.ds(half_base + c * CCHUNK, CCHUNK), :] = (
                jax.lax.dot_general(
                    av.astype(jnp.bfloat16), wv_bf, (((1,), (0,)), ((), ())),
                    preferred_element_type=jnp.float32,
                )
            )

        def compute_c8(c8, b_base, half_base):
            compute_chunk(2 * c8, b_base, half_base)
            compute_chunk(2 * c8 + 1, b_base, half_base)

        def chunk8(i):
            return lax.rem(q + i + 2 * N8, N8)

        def rd(src, dst, send_sem, recv_sem, dev):
            return pltpu.make_async_remote_copy(
                src_ref=src, dst_ref=dst, send_sem=send_sem,
                recv_sem=recv_sem, device_id=(dev,),
                device_id_type=pl.DeviceIdType.MESH,
            )

        def out_chunk(off):
            return out_ref[pl.ds(off, CHUNK8), :]

        compute_c8(chunk8(0), 0, 0)
        compute_c8(chunk8(0), 2, HALF)

        pending_sends = []

        cw_sb[0, :, :] = out_chunk(chunk8(0) * CHUNK8).astype(jnp.bfloat16)
        ccw_sb[0, :, :] = out_chunk(
            HALF + chunk8(0) * CHUNK8).astype(jnp.bfloat16)

        for s in range(HOPS8):
            if DO_RS:
                cw = rd(cw_sb.at[s], cw_stage.at[s],
                        cw_rs_send.at[s], cw_rs_recv.at[s], right)
                ccw = rd(ccw_sb.at[s], ccw_stage.at[s],
                         ccw_rs_send.at[s], ccw_rs_recv.at[s], left)
                cw.start()
                ccw.start()
                pending_sends += [cw, ccw]
            compute_c8(chunk8(-s - 1), 0, 0)
            compute_c8(chunk8(s + 1), 2, HALF)
            if not DO_RS:
                continue
            cw.wait_recv()
            off = chunk8(-s - 1) * CHUNK8
            summed = out_chunk(off) + cw_stage[s, :, :].astype(jnp.float32)
            out_ref[pl.ds(off, CHUNK8), :] = summed
            if s < HOPS8 - 1:
                cw_sb[s + 1, :, :] = summed.astype(jnp.bfloat16)
            else:
                cw_exsb[:, :] = summed.astype(jnp.bfloat16)
                cwx = [rd(cw_exsb.at[pl.ds(u * SUB, SUB), :],
                          cw_exst.at[pl.ds(u * SUB, SUB), :],
                          ex_send.at[u], ex_recv.at[u], partner)
                       for u in range(2)]
                for r in cwx:
                    r.start()
                pending_sends += cwx
            ccw.wait_recv()
            off = HALF + chunk8(s + 1) * CHUNK8
            summed = out_chunk(off) + ccw_stage[s, :, :].astype(jnp.float32)
            out_ref[pl.ds(off, CHUNK8), :] = summed
            if s < HOPS8 - 1:
                ccw_sb[s + 1, :, :] = summed.astype(jnp.bfloat16)
            else:
                ccw_exsb[:, :] = summed.astype(jnp.bfloat16)
                ccwx = [rd(ccw_exsb.at[pl.ds(u * SUB, SUB), :],
                           ccw_exst.at[pl.ds(u * SUB, SUB), :],
                           ex_send.at[2 + u], ex_recv.at[2 + u], partner)
                        for u in range(2)]
                for r in ccwx:
                    r.start()
                pending_sends += ccwx

        ag_sems = [(cw1_ag_send, cw1_ag_recv), (cw2_ag_send, cw2_ag_recv),
                   (ccw1_ag_send, ccw1_ag_recv), (ccw2_ag_send, ccw2_ag_recv)]

        def ag_rd(sysid, buf, base_off, s, dev):
            off = base_off + (sysid % 2) * SUB
            send_sems, recv_sems = ag_sems[sysid]
            return rd(buf.at[pl.ds(off, SUB), :],
                      buf.at[pl.ds(off, SUB), :],
                      send_sems.at[s], recv_sems.at[s], dev)

        prev = [None] * 4
        if DO_RS:
            off_cw = chunk8(1) * CHUNK8
            off_ccw = chunk8(-1) * CHUNK8
            for u in range(2):
                cwx[u].wait_recv()
                o = off_cw + u * SUB
                fin = (out_ref[pl.ds(o, SUB), :]
                       + cw_exst[pl.ds(u * SUB, SUB), :].astype(jnp.float32))
                out_ref[pl.ds(o, SUB), :] = fin
                cw_ag[pl.ds(o, SUB), :] = fin.astype(jnp.bfloat16)
                if DO_AG:
                    prev[u] = ag_rd(u, cw_ag, off_cw, 0, right)
                    prev[u].start()
            for u in range(2):
                ccwx[u].wait_recv()
                o = off_ccw + u * SUB
                fin = (out_ref[pl.ds(HALF + o, SUB), :]
                       + ccw_exst[pl.ds(u * SUB, SUB), :].astype(jnp.float32))
                out_ref[pl.ds(HALF + o, SUB), :] = fin
                ccw_ag[pl.ds(o, SUB), :] = fin.astype(jnp.bfloat16)
                if DO_AG:
                    prev[2 + u] = ag_rd(2 + u, ccw_ag, off_ccw, 0, left)
                    prev[2 + u].start()
            if DO_AG:
                pending_sends += prev

        for s in range(1, HOPS8 if DO_AG else 1):
            cw_off = chunk8(1 - s) * CHUNK8
            ccw_off = chunk8(s - 1) * CHUNK8
            for i, (buf, base, dev) in enumerate(
                    [(cw_ag, cw_off, right), (cw_ag, cw_off, right),
                     (ccw_ag, ccw_off, left), (ccw_ag, ccw_off, left)]):
                prev[i].wait_recv()
                prev[i] = ag_rd(i, buf, base, s, dev)
                prev[i].start()
            pending_sends += prev
            out_ref[pl.ds(cw_off, CHUNK8), :] = (
                cw_ag[pl.ds(cw_off, CHUNK8), :].astype(jnp.float32))
            out_ref[pl.ds(HALF + ccw_off, CHUNK8), :] = (
                ccw_ag[pl.ds(ccw_off, CHUNK8), :].astype(jnp.float32))
        if DO_AG:
            for r in prev:
                r.wait_recv()
            o = chunk8(-HOPS8 + 1) * CHUNK8
            out_ref[pl.ds(o, CHUNK8), :] = (
                cw_ag[pl.ds(o, CHUNK8), :].astype(jnp.float32))
            o = chunk8(HOPS8 - 1) * CHUNK8
            out_ref[pl.ds(HALF + o, CHUNK8), :] = (
                ccw_ag[pl.ds(o, CHUNK8), :].astype(jnp.float32))

        for r in pending_sends:
            r.wait_send()

    out = pl.pallas_call(
        body,
        out_shape=jax.ShapeDtypeStruct((ROWS, C_OUT), jnp.float32),
        in_specs=[pl.BlockSpec(memory_space=pltpu.VMEM)] * 3,
        out_specs=pl.BlockSpec(memory_space=pltpu.VMEM),
        scratch_shapes=[
            pltpu.VMEM((HOPS8, CHUNK8, C_OUT), jnp.bfloat16),
            pltpu.VMEM((HOPS8, CHUNK8, C_OUT), jnp.bfloat16),
            pltpu.VMEM((HOPS8, CHUNK8, C_OUT), jnp.bfloat16),
            pltpu.VMEM((HOPS8, CHUNK8, C_OUT), jnp.bfloat16),
            pltpu.VMEM((CHUNK8, C_OUT), jnp.bfloat16),
            pltpu.VMEM((CHUNK8, C_OUT), jnp.bfloat16),
            pltpu.VMEM((CHUNK8, C_OUT), jnp.bfloat16),
            pltpu.VMEM((CHUNK8, C_OUT), jnp.bfloat16),
            pltpu.VMEM((HALF, C_OUT), jnp.bfloat16),
            pltpu.VMEM((HALF, C_OUT), jnp.bfloat16),
        ] + [pltpu.SemaphoreType.DMA((HOPS8,))] * 12
          + [pltpu.SemaphoreType.DMA((4,))] * 2,
        compiler_params=pltpu.CompilerParams(collective_id=0),
    )(x, k, Wp)
    return out.reshape(B, S, C_OUT)


# device time: 76723 ns/iter; 1.1366x vs baseline; 1.0007x over previous
import os

import jax
import jax.numpy as jnp
from jax import lax
from jax.experimental import pallas as pl
from jax.experimental.pallas import tpu as pltpu

ABLATE = int(os.environ.get("ABLATE", "0"))
DO_RS = ABLATE < 2
DO_AG = ABLATE < 1

N_DEV = 16
B, S, C_IN, C_OUT = 4, 1024, 512, 512
ROWS = B * S
HALF = ROWS // 2
N8 = 8
HOPS8 = N8 - 1
CHUNK8 = HALF // N8
SUB = CHUNK8 // 2
CCHUNK = 128
CPB = S // CCHUNK

RING_A = [0, 4, 8, 12, 15, 11, 7, 3]
RING_B = [1, 5, 9, 13, 14, 10, 6, 2]
Q = [0] * N_DEV
RIGHT8 = [0] * N_DEV
LEFT8 = [0] * N_DEV
PARTNER = [0] * N_DEV
for _i in range(N8):
    _a, _b = RING_A[_i], RING_B[_i]
    Q[_a] = Q[_b] = _i
    RIGHT8[_a], LEFT8[_a] = RING_A[(_i + 1) % N8], RING_A[(_i - 1) % N8]
    RIGHT8[_b], LEFT8[_b] = RING_B[(_i + 1) % N8], RING_B[(_i - 1) % N8]
    PARTNER[_a], PARTNER[_b] = _b, _a


def _lut(table, idx):
    acc = jnp.int32(table[0])
    for i in range(1, len(table)):
        acc = jnp.where(idx == i, jnp.int32(table[i]), acc)
    return acc


def kernel(x, k, Wp):
    def body(x_ref, k_ref, w_ref, out_ref,
             cw_sb, ccw_sb, cw_stage, ccw_stage,
             cw_exsb, ccw_exsb, cw_exst, ccw_exst,
             cw_ag, ccw_ag,
             cw_rs_send, cw_rs_recv, ccw_rs_send, ccw_rs_recv,
             cw1_ag_send, cw1_ag_recv, cw2_ag_send, cw2_ag_recv,
             ccw1_ag_send, ccw1_ag_recv, ccw2_ag_send, ccw2_ag_recv,
             ex_send, ex_recv):
        my = lax.axis_index("i")
        q = _lut(Q, my)
        right = _lut(RIGHT8, my)
        left = _lut(LEFT8, my)
        partner = _lut(PARTNER, my)

        barrier = pltpu.get_barrier_semaphore()
        for nbr in (left, right, partner):
            pl.semaphore_signal(barrier, inc=1, device_id=(nbr,),
                                device_id_type=pl.DeviceIdType.MESH)
        pl.semaphore_wait(barrier, 3)

        kv = k_ref[:, :]
        wv_bf = w_ref[:, :].astype(jnp.bfloat16)

        def compute_c8(c8, b_base, half_base):
            b = lax.div(c8, S // CHUNK8) + b_base
            s0 = pl.multiple_of(lax.rem(c8, S // CHUNK8) * CHUNK8, CHUNK8)
            xc = x_ref[b, pl.ds(s0, CHUNK8), :]
            hs = pl.multiple_of(jnp.maximum(s0 - 8, 0), 8)
            halo = x_ref[b, pl.ds(hs, 8), :][5:8]
            halo = jnp.where(s0 == 0, jnp.zeros_like(halo), halo)
            xe = jnp.concatenate([halo, xc], axis=0)
            accv = xe[3:3 + CHUNK8] * kv[3][None, :]
            for t in range(3):
                accv = accv + xe[t:t + CHUNK8] * kv[t][None, :]
            av = accv / (1.0 + jnp.exp(-accv))
            out_ref[pl.ds(half_base + c8 * CHUNK8, CHUNK8), :] = (
                jax.lax.dot_general(
                    av.astype(jnp.bfloat16), wv_bf, (((1,), (0,)), ((), ())),
                    preferred_element_type=jnp.float32,
                )
            )

        def chunk8(i):
            return lax.rem(q + i + 2 * N8, N8)

        def rd(src, dst, send_sem, recv_sem, dev):
            return pltpu.make_async_remote_copy(
                src_ref=src, dst_ref=dst, send_sem=send_sem,
                recv_sem=recv_sem, device_id=(dev,),
                device_id_type=pl.DeviceIdType.MESH,
            )

        def out_chunk(off):
            return out_ref[pl.ds(off, CHUNK8), :]

        compute_c8(chunk8(0), 0, 0)
        compute_c8(chunk8(0), 2, HALF)

        pending_sends = []

        cw_sb[0, :, :] = out_chunk(chunk8(0) * CHUNK8).astype(jnp.bfloat16)
        ccw_sb[0, :, :] = out_chunk(
            HALF + chunk8(0) * CHUNK8).astype(jnp.bfloat16)

        for s in range(HOPS8):
            if DO_RS:
                cw = rd(cw_sb.at[s], cw_stage.at[s],
                        cw_rs_send.at[s], cw_rs_recv.at[s], right)
                ccw = rd(ccw_sb.at[s], ccw_stage.at[s],
                         ccw_rs_send.at[s], ccw_rs_recv.at[s], left)
                cw.start()
                ccw.start()
                pending_sends += [cw, ccw]
            compute_c8(chunk8(-s - 1), 0, 0)
            compute_c8(chunk8(s + 1), 2, HALF)
            if not DO_RS:
                continue
            cw.wait_recv()
            off = chunk8(-s - 1) * CHUNK8
            summed = out_chunk(off) + cw_stage[s, :, :].astype(jnp.float32)
            out_ref[pl.ds(off, CHUNK8), :] = summed
            if s < HOPS8 - 1:
                cw_sb[s + 1, :, :] = summed.astype(jnp.bfloat16)
            else:
                cw_exsb[:, :] = summed.astype(jnp.bfloat16)
                cwx = [rd(cw_exsb.at[pl.ds(u * SUB, SUB), :],
                          cw_exst.at[pl.ds(u * SUB, SUB), :],
                          ex_send.at[u], ex_recv.at[u], partner)
                       for u in range(2)]
                for r in cwx:
                    r.start()
                pending_sends += cwx
            ccw.wait_recv()
            off = HALF + chunk8(s + 1) * CHUNK8
            summed = out_chunk(off) + ccw_stage[s, :, :].astype(jnp.float32)
            out_ref[pl.ds(off, CHUNK8), :] = summed
            if s < HOPS8 - 1:
                ccw_sb[s + 1, :, :] = summed.astype(jnp.bfloat16)
            else:
                ccw_exsb[:, :] = summed.astype(jnp.bfloat16)
                ccwx = [rd(ccw_exsb.at[pl.ds(u * SUB, SUB), :],
                           ccw_exst.at[pl.ds(u * SUB, SUB), :],
                           ex_send.at[2 + u], ex_recv.at[2 + u], partner)
                        for u in range(2)]
                for r in ccwx:
                    r.start()
                pending_sends += ccwx

        ag_sems = [(cw1_ag_send, cw1_ag_recv), (cw2_ag_send, cw2_ag_recv),
                   (ccw1_ag_send, ccw1_ag_recv), (ccw2_ag_send, ccw2_ag_recv)]

        def ag_rd(sysid, buf, base_off, s, dev):
            off = base_off + (sysid % 2) * SUB
            send_sems, recv_sems = ag_sems[sysid]
            return rd(buf.at[pl.ds(off, SUB), :],
                      buf.at[pl.ds(off, SUB), :],
                      send_sems.at[s], recv_sems.at[s], dev)

        prev = [None] * 4
        if DO_RS:
            off_cw = chunk8(1) * CHUNK8
            off_ccw = chunk8(-1) * CHUNK8
            for u in range(2):
                cwx[u].wait_recv()
                o = off_cw + u * SUB
                fin = (out_ref[pl.ds(o, SUB), :]
                       + cw_exst[pl.ds(u * SUB, SUB), :].astype(jnp.float32))
                out_ref[pl.ds(o, SUB), :] = fin
                cw_ag[pl.ds(o, SUB), :] = fin.astype(jnp.bfloat16)
                if DO_AG:
                    prev[u] = ag_rd(u, cw_ag, off_cw, 0, right)
                    prev[u].start()
            for u in range(2):
                ccwx[u].wait_recv()
                o = off_ccw + u * SUB
                fin = (out_ref[pl.ds(HALF + o, SUB), :]
                       + ccw_exst[pl.ds(u * SUB, SUB), :].astype(jnp.float32))
                out_ref[pl.ds(HALF + o, SUB), :] = fin
                ccw_ag[pl.ds(o, SUB), :] = fin.astype(jnp.bfloat16)
                if DO_AG:
                    prev[2 + u] = ag_rd(2 + u, ccw_ag, off_ccw, 0, left)
                    prev[2 + u].start()
            if DO_AG:
                pending_sends += prev

        for s in range(1, HOPS8 if DO_AG else 1):
            cw_off = chunk8(1 - s) * CHUNK8
            ccw_off = chunk8(s - 1) * CHUNK8
            for i, (buf, base, dev) in enumerate(
                    [(cw_ag, cw_off, right), (cw_ag, cw_off, right),
                     (ccw_ag, ccw_off, left), (ccw_ag, ccw_off, left)]):
                prev[i].wait_recv()
                prev[i] = ag_rd(i, buf, base, s, dev)
                prev[i].start()
            pending_sends += prev
            out_ref[pl.ds(cw_off, CHUNK8), :] = (
                cw_ag[pl.ds(cw_off, CHUNK8), :].astype(jnp.float32))
            out_ref[pl.ds(HALF + ccw_off, CHUNK8), :] = (
                ccw_ag[pl.ds(ccw_off, CHUNK8), :].astype(jnp.float32))
        if DO_AG:
            for r in prev:
                r.wait_recv()
            o = chunk8(-HOPS8 + 1) * CHUNK8
            out_ref[pl.ds(o, CHUNK8), :] = (
                cw_ag[pl.ds(o, CHUNK8), :].astype(jnp.float32))
            o = chunk8(HOPS8 - 1) * CHUNK8
            out_ref[pl.ds(HALF + o, CHUNK8), :] = (
                ccw_ag[pl.ds(o, CHUNK8), :].astype(jnp.float32))

        for r in pending_sends:
            r.wait_send()

    out = pl.pallas_call(
        body,
        out_shape=jax.ShapeDtypeStruct((ROWS, C_OUT), jnp.float32),
        in_specs=[pl.BlockSpec(memory_space=pltpu.VMEM)] * 3,
        out_specs=pl.BlockSpec(memory_space=pltpu.VMEM),
        scratch_shapes=[
            pltpu.VMEM((HOPS8, CHUNK8, C_OUT), jnp.bfloat16),
            pltpu.VMEM((HOPS8, CHUNK8, C_OUT), jnp.bfloat16),
            pltpu.VMEM((HOPS8, CHUNK8, C_OUT), jnp.bfloat16),
            pltpu.VMEM((HOPS8, CHUNK8, C_OUT), jnp.bfloat16),
            pltpu.VMEM((CHUNK8, C_OUT), jnp.bfloat16),
            pltpu.VMEM((CHUNK8, C_OUT), jnp.bfloat16),
            pltpu.VMEM((CHUNK8, C_OUT), jnp.bfloat16),
            pltpu.VMEM((CHUNK8, C_OUT), jnp.bfloat16),
            pltpu.VMEM((HALF, C_OUT), jnp.bfloat16),
            pltpu.VMEM((HALF, C_OUT), jnp.bfloat16),
        ] + [pltpu.SemaphoreType.DMA((HOPS8,))] * 12
          + [pltpu.SemaphoreType.DMA((4,))] * 2,
        compiler_params=pltpu.CompilerParams(collective_id=0),
    )(x, k, Wp)
    return out.reshape(B, S, C_OUT)


# device time: 76568 ns/iter; 1.1389x vs baseline; 1.0020x over previous
import os

import jax
import jax.numpy as jnp
from jax import lax
from jax.experimental import pallas as pl
from jax.experimental.pallas import tpu as pltpu

ABLATE = int(os.environ.get("ABLATE", "0"))
DO_RS = ABLATE != 2
DO_AG = ABLATE not in (1, 2)
DO_COMPUTE = ABLATE != 4

N_DEV = 16
B, S, C_IN, C_OUT = 4, 1024, 512, 512
ROWS = B * S
HALF = ROWS // 2
N8 = 8
HOPS8 = N8 - 1
CHUNK8 = HALF // N8
SUB = CHUNK8 // 2
CCHUNK = 128
CPB = S // CCHUNK

RING_A = [0, 4, 8, 12, 15, 11, 7, 3]
RING_B = [1, 5, 9, 13, 14, 10, 6, 2]
Q = [0] * N_DEV
RIGHT8 = [0] * N_DEV
LEFT8 = [0] * N_DEV
PARTNER = [0] * N_DEV
for _i in range(N8):
    _a, _b = RING_A[_i], RING_B[_i]
    Q[_a] = Q[_b] = _i
    RIGHT8[_a], LEFT8[_a] = RING_A[(_i + 1) % N8], RING_A[(_i - 1) % N8]
    RIGHT8[_b], LEFT8[_b] = RING_B[(_i + 1) % N8], RING_B[(_i - 1) % N8]
    PARTNER[_a], PARTNER[_b] = _b, _a


def _lut(table, idx):
    acc = jnp.int32(table[0])
    for i in range(1, len(table)):
        acc = jnp.where(idx == i, jnp.int32(table[i]), acc)
    return acc


def kernel(x, k, Wp):
    def body(x_ref, k_ref, w_ref, out_ref,
             cw_stage, ccw_stage, cw_exst, ccw_exst,
             cw_ag, ccw_ag,
             cw_rs_send, cw_rs_recv, ccw_rs_send, ccw_rs_recv,
             cw1_ag_send, cw1_ag_recv, cw2_ag_send, cw2_ag_recv,
             ccw1_ag_send, ccw1_ag_recv, ccw2_ag_send, ccw2_ag_recv,
             ex_send, ex_recv):
        my = lax.axis_index("i")
        q = _lut(Q, my)
        right = _lut(RIGHT8, my)
        left = _lut(LEFT8, my)
        partner = _lut(PARTNER, my)

        barrier = pltpu.get_barrier_semaphore()
        for nbr in (left, right, partner):
            pl.semaphore_signal(barrier, inc=1, device_id=(nbr,),
                                device_id_type=pl.DeviceIdType.MESH)
        pl.semaphore_wait(barrier, 3)

        kv = k_ref[:, :]
        wv_bf = w_ref[:, :].astype(jnp.bfloat16)

        def compute_c8(c8, b_base, buf):
            b = lax.div(c8, S // CHUNK8) + b_base
            s0 = pl.multiple_of(lax.rem(c8, S // CHUNK8) * CHUNK8, CHUNK8)
            xc = x_ref[b, pl.ds(s0, CHUNK8), :]
            hs = pl.multiple_of(jnp.maximum(s0 - 8, 0), 8)
            halo = x_ref[b, pl.ds(hs, 8), :][5:8]
            halo = jnp.where(s0 == 0, jnp.zeros_like(halo), halo)
            xe = jnp.concatenate([halo, xc], axis=0)
            accv = xe[3:3 + CHUNK8] * kv[3][None, :]
            for t in range(3):
                accv = accv + xe[t:t + CHUNK8] * kv[t][None, :]
            av = accv / (1.0 + jnp.exp(-accv))
            buf[pl.ds(c8 * CHUNK8, CHUNK8), :] = (
                jax.lax.dot_general(
                    av.astype(jnp.bfloat16), wv_bf, (((1,), (0,)), ((), ())),
                    preferred_element_type=jnp.float32,
                ).astype(jnp.bfloat16)
            )

        def chunk8(i):
            return lax.rem(q + i + 2 * N8, N8)

        def rd(src, dst, send_sem, recv_sem, dev):
            return pltpu.make_async_remote_copy(
                src_ref=src, dst_ref=dst, send_sem=send_sem,
                recv_sem=recv_sem, device_id=(dev,),
                device_id_type=pl.DeviceIdType.MESH,
            )

        def out_chunk(off):
            return out_ref[pl.ds(off, CHUNK8), :]

        if DO_COMPUTE:
            compute_c8(chunk8(0), 0, cw_ag)
            compute_c8(chunk8(0), 2, ccw_ag)

        pending_sends = []

        for s in range(HOPS8):
            if DO_RS:
                o_send = chunk8(-s) * CHUNK8
                cw = rd(cw_ag.at[pl.ds(o_send, CHUNK8), :], cw_stage.at[s],
                        cw_rs_send.at[s], cw_rs_recv.at[s], right)
                o_send = chunk8(s) * CHUNK8
                ccw = rd(ccw_ag.at[pl.ds(o_send, CHUNK8), :],
                         ccw_stage.at[s],
                         ccw_rs_send.at[s], ccw_rs_recv.at[s], left)
                cw.start()
                ccw.start()
                pending_sends += [cw, ccw]
            if DO_COMPUTE:
                compute_c8(chunk8(-s - 1), 0, cw_ag)
                compute_c8(chunk8(s + 1), 2, ccw_ag)
            if not DO_RS:
                continue
            cw.wait_recv()
            off = chunk8(-s - 1) * CHUNK8
            summed = (cw_ag[pl.ds(off, CHUNK8), :].astype(jnp.float32)
                      + cw_stage[s, :, :].astype(jnp.float32))
            cw_ag[pl.ds(off, CHUNK8), :] = summed.astype(jnp.bfloat16)
            if s == HOPS8 - 1:
                out_ref[pl.ds(off, CHUNK8), :] = summed
                cwx = [rd(cw_ag.at[pl.ds(off + u * SUB, SUB), :],
                          cw_exst.at[pl.ds(u * SUB, SUB), :],
                          ex_send.at[u], ex_recv.at[u], partner)
                       for u in range(2)]
                for r in cwx:
                    r.start()
            ccw.wait_recv()
            off = chunk8(s + 1) * CHUNK8
            summed = (ccw_ag[pl.ds(off, CHUNK8), :].astype(jnp.float32)
                      + ccw_stage[s, :, :].astype(jnp.float32))
            ccw_ag[pl.ds(off, CHUNK8), :] = summed.astype(jnp.bfloat16)
            if s == HOPS8 - 1:
                out_ref[pl.ds(HALF + off, CHUNK8), :] = summed
                ccwx = [rd(ccw_ag.at[pl.ds(off + u * SUB, SUB), :],
                           ccw_exst.at[pl.ds(u * SUB, SUB), :],
                           ex_send.at[2 + u], ex_recv.at[2 + u], partner)
                        for u in range(2)]
                for r in ccwx:
                    r.start()

        ag_sems = [(cw1_ag_send, cw1_ag_recv), (cw2_ag_send, cw2_ag_recv),
                   (ccw1_ag_send, ccw1_ag_recv), (ccw2_ag_send, ccw2_ag_recv)]

        def ag_rd(sysid, buf, base_off, s, dev):
            off = base_off + (sysid % 2) * SUB
            send_sems, recv_sems = ag_sems[sysid]
            return rd(buf.at[pl.ds(off, SUB), :],
                      buf.at[pl.ds(off, SUB), :],
                      send_sems.at[s], recv_sems.at[s], dev)

        prev = [None] * 4
        if DO_RS:
            off_cw = chunk8(1) * CHUNK8
            off_ccw = chunk8(-1) * CHUNK8
            for u in range(2):
                cwx[u].wait_recv()
                cwx[u].wait_send()
                o = off_cw + u * SUB
                fin = (out_ref[pl.ds(o, SUB), :]
                       + cw_exst[pl.ds(u * SUB, SUB), :].astype(jnp.float32))
                out_ref[pl.ds(o, SUB), :] = fin
                cw_ag[pl.ds(o, SUB), :] = fin.astype(jnp.bfloat16)
                if DO_AG:
                    prev[u] = ag_rd(u, cw_ag, off_cw, 0, right)
                    prev[u].start()
            for u in range(2):
                ccwx[u].wait_recv()
                ccwx[u].wait_send()
                o = off_ccw + u * SUB
                fin = (out_ref[pl.ds(HALF + o, SUB), :]
                       + ccw_exst[pl.ds(u * SUB, SUB), :].astype(jnp.float32))
                out_ref[pl.ds(HALF + o, SUB), :] = fin
                ccw_ag[pl.ds(o, SUB), :] = fin.astype(jnp.bfloat16)
                if DO_AG:
                    prev[2 + u] = ag_rd(2 + u, ccw_ag, off_ccw, 0, left)
                    prev[2 + u].start()
            if DO_AG:
                pending_sends += prev

        for s in range(1, HOPS8 if DO_AG else 1):
            cw_off = chunk8(1 - s) * CHUNK8
            ccw_off = chunk8(s - 1) * CHUNK8
            for i, (buf, base, dev) in enumerate(
                    [(cw_ag, cw_off, right), (cw_ag, cw_off, right),
                     (ccw_ag, ccw_off, left), (ccw_ag, ccw_off, left)]):
                prev[i].wait_recv()
                prev[i] = ag_rd(i, buf, base, s, dev)
                prev[i].start()
            pending_sends += prev
            out_ref[pl.ds(cw_off, CHUNK8), :] = (
                cw_ag[pl.ds(cw_off, CHUNK8), :].astype(jnp.float32))
            out_ref[pl.ds(HALF + ccw_off, CHUNK8), :] = (
                ccw_ag[pl.ds(ccw_off, CHUNK8), :].astype(jnp.float32))
        if DO_AG:
            for r in prev:
                r.wait_recv()
            o = chunk8(-HOPS8 + 1) * CHUNK8
            out_ref[pl.ds(o, CHUNK8), :] = (
                cw_ag[pl.ds(o, CHUNK8), :].astype(jnp.float32))
            o = chunk8(HOPS8 - 1) * CHUNK8
            out_ref[pl.ds(HALF + o, CHUNK8), :] = (
                ccw_ag[pl.ds(o, CHUNK8), :].astype(jnp.float32))

        for r in pending_sends:
            r.wait_send()

    out = pl.pallas_call(
        body,
        out_shape=jax.ShapeDtypeStruct((ROWS, C_OUT), jnp.float32),
        in_specs=[pl.BlockSpec(memory_space=pltpu.VMEM)] * 3,
        out_specs=pl.BlockSpec(memory_space=pltpu.VMEM),
        scratch_shapes=[
            pltpu.VMEM((HOPS8, CHUNK8, C_OUT), jnp.bfloat16),
            pltpu.VMEM((HOPS8, CHUNK8, C_OUT), jnp.bfloat16),
            pltpu.VMEM((CHUNK8, C_OUT), jnp.bfloat16),
            pltpu.VMEM((CHUNK8, C_OUT), jnp.bfloat16),
            pltpu.VMEM((HALF, C_OUT), jnp.bfloat16),
            pltpu.VMEM((HALF, C_OUT), jnp.bfloat16),
        ] + [pltpu.SemaphoreType.DMA((HOPS8,))] * 12
          + [pltpu.SemaphoreType.DMA((4,))] * 2,
        compiler_params=pltpu.CompilerParams(collective_id=0),
    )(x, k, Wp)
    return out.reshape(B, S, C_OUT)


# device time: 74995 ns/iter; 1.1628x vs baseline; 1.0210x over previous
import os

import jax
import jax.numpy as jnp
from jax import lax
from jax.experimental import pallas as pl
from jax.experimental.pallas import tpu as pltpu

ABLATE = int(os.environ.get("ABLATE", "0"))
DO_RS = ABLATE != 2
DO_AG = ABLATE not in (1, 2)
DO_COMPUTE = ABLATE != 4

N_DEV = 16
B, S, C_IN, C_OUT = 4, 1024, 512, 512
ROWS = B * S
HALF = ROWS // 2
N8 = 8
HOPS8 = N8 - 1
CHUNK8 = HALF // N8
SUB = CHUNK8 // 2
CCHUNK = 128
CPB = S // CCHUNK

RING_A = [0, 4, 8, 12, 15, 11, 7, 3]
RING_B = [1, 5, 9, 13, 14, 10, 6, 2]
Q = [0] * N_DEV
RIGHT8 = [0] * N_DEV
LEFT8 = [0] * N_DEV
PARTNER = [0] * N_DEV
for _i in range(N8):
    _a, _b = RING_A[_i], RING_B[_i]
    Q[_a] = Q[_b] = _i
    RIGHT8[_a], LEFT8[_a] = RING_A[(_i + 1) % N8], RING_A[(_i - 1) % N8]
    RIGHT8[_b], LEFT8[_b] = RING_B[(_i + 1) % N8], RING_B[(_i - 1) % N8]
    PARTNER[_a], PARTNER[_b] = _b, _a


def _lut(table, idx):
    acc = jnp.int32(table[0])
    for i in range(1, len(table)):
        acc = jnp.where(idx == i, jnp.int32(table[i]), acc)
    return acc


def kernel(x, k, Wp):
    def body(x_ref, k_ref, w_ref, out_ref,
             cw_stage, ccw_stage, cw_exst, ccw_exst,
             cw_ag, ccw_ag,
             cw1_rs_send, cw1_rs_recv, cw2_rs_send, cw2_rs_recv,
             ccw1_rs_send, ccw1_rs_recv, ccw2_rs_send, ccw2_rs_recv,
             cw1_ag_send, cw1_ag_recv, cw2_ag_send, cw2_ag_recv,
             ccw1_ag_send, ccw1_ag_recv, ccw2_ag_send, ccw2_ag_recv,
             ex_send, ex_recv):
        my = lax.axis_index("i")
        q = _lut(Q, my)
        right = _lut(RIGHT8, my)
        left = _lut(LEFT8, my)
        partner = _lut(PARTNER, my)

        barrier = pltpu.get_barrier_semaphore()
        for nbr in (left, right, partner):
            pl.semaphore_signal(barrier, inc=1, device_id=(nbr,),
                                device_id_type=pl.DeviceIdType.MESH)
        pl.semaphore_wait(barrier, 3)

        kv = k_ref[:, :]
        wv_bf = w_ref[:, :].astype(jnp.bfloat16)

        def compute_c8(c8, b_base, buf):
            b = lax.div(c8, S // CHUNK8) + b_base
            s0 = pl.multiple_of(lax.rem(c8, S // CHUNK8) * CHUNK8, CHUNK8)
            xc = x_ref[b, pl.ds(s0, CHUNK8), :]
            hs = pl.multiple_of(jnp.maximum(s0 - 8, 0), 8)
            halo = x_ref[b, pl.ds(hs, 8), :][5:8]
            halo = jnp.where(s0 == 0, jnp.zeros_like(halo), halo)
            xe = jnp.concatenate([halo, xc], axis=0)
            accv = xe[3:3 + CHUNK8] * kv[3][None, :]
            for t in range(3):
                accv = accv + xe[t:t + CHUNK8] * kv[t][None, :]
            av = accv / (1.0 + jnp.exp(-accv))
            buf[pl.ds(c8 * CHUNK8, CHUNK8), :] = (
                jax.lax.dot_general(
                    av.astype(jnp.bfloat16), wv_bf, (((1,), (0,)), ((), ())),
                    preferred_element_type=jnp.float32,
                ).astype(jnp.bfloat16)
            )

        def chunk8(i):
            return lax.rem(q + i + 2 * N8, N8)

        def rd(src, dst, send_sem, recv_sem, dev):
            return pltpu.make_async_remote_copy(
                src_ref=src, dst_ref=dst, send_sem=send_sem,
                recv_sem=recv_sem, device_id=(dev,),
                device_id_type=pl.DeviceIdType.MESH,
            )

        def out_chunk(off):
            return out_ref[pl.ds(off, CHUNK8), :]

        if DO_COMPUTE:
            compute_c8(chunk8(0), 0, cw_ag)
            compute_c8(chunk8(0), 2, ccw_ag)

        pending_sends = []

        rs_sems = [(cw1_rs_send, cw1_rs_recv), (cw2_rs_send, cw2_rs_recv),
                   (ccw1_rs_send, ccw1_rs_recv), (ccw2_rs_send, ccw2_rs_recv)]

        cwx = [None] * 2
        ccwx = [None] * 2
        for s in range(HOPS8):
            if DO_RS:
                o_cw = chunk8(-s) * CHUNK8
                o_ccw = chunk8(s) * CHUNK8
                rs = [rd(cw_ag.at[pl.ds(o_cw + u * SUB, SUB), :],
                         cw_stage.at[s, pl.ds(u * SUB, SUB), :],
                         rs_sems[u][0].at[s], rs_sems[u][1].at[s], right)
                      for u in range(2)]
                rs += [rd(ccw_ag.at[pl.ds(o_ccw + u * SUB, SUB), :],
                          ccw_stage.at[s, pl.ds(u * SUB, SUB), :],
                          rs_sems[2 + u][0].at[s], rs_sems[2 + u][1].at[s],
                          left)
                       for u in range(2)]
                for r in rs:
                    r.start()
                pending_sends += rs
            if DO_COMPUTE:
                compute_c8(chunk8(-s - 1), 0, cw_ag)
                compute_c8(chunk8(s + 1), 2, ccw_ag)
            if not DO_RS:
                continue
            off_cw = chunk8(-s - 1) * CHUNK8
            off_ccw = chunk8(s + 1) * CHUNK8
            for u in range(2):
                rs[u].wait_recv()
                o = off_cw + u * SUB
                summed = (cw_ag[pl.ds(o, SUB), :].astype(jnp.float32)
                          + cw_stage[s, pl.ds(u * SUB, SUB), :]
                          .astype(jnp.float32))
                cw_ag[pl.ds(o, SUB), :] = summed.astype(jnp.bfloat16)
                if s == HOPS8 - 1:
                    out_ref[pl.ds(o, SUB), :] = summed
                    cwx[u] = rd(cw_ag.at[pl.ds(o, SUB), :],
                                cw_exst.at[pl.ds(u * SUB, SUB), :],
                                ex_send.at[u], ex_recv.at[u], partner)
                    cwx[u].start()
            for u in range(2):
                rs[2 + u].wait_recv()
                o = off_ccw + u * SUB
                summed = (ccw_ag[pl.ds(o, SUB), :].astype(jnp.float32)
                          + ccw_stage[s, pl.ds(u * SUB, SUB), :]
                          .astype(jnp.float32))
                ccw_ag[pl.ds(o, SUB), :] = summed.astype(jnp.bfloat16)
                if s == HOPS8 - 1:
                    out_ref[pl.ds(HALF + o, SUB), :] = summed
                    ccwx[u] = rd(ccw_ag.at[pl.ds(o, SUB), :],
                                 ccw_exst.at[pl.ds(u * SUB, SUB), :],
                                 ex_send.at[2 + u], ex_recv.at[2 + u],
                                 partner)
                    ccwx[u].start()

        ag_sems = [(cw1_ag_send, cw1_ag_recv), (cw2_ag_send, cw2_ag_recv),
                   (ccw1_ag_send, ccw1_ag_recv), (ccw2_ag_send, ccw2_ag_recv)]

        def ag_rd(sysid, buf, base_off, s, dev):
            off = base_off + (sysid % 2) * SUB
            send_sems, recv_sems = ag_sems[sysid]
            return rd(buf.at[pl.ds(off, SUB), :],
                      buf.at[pl.ds(off, SUB), :],
                      send_sems.at[s], recv_sems.at[s], dev)

        prev = [None] * 4
        if DO_RS:
            off_cw = chunk8(1) * CHUNK8
            off_ccw = chunk8(-1) * CHUNK8
            for u in range(2):
                cwx[u].wait_recv()
                cwx[u].wait_send()
                o = off_cw + u * SUB
                fin = (out_ref[pl.ds(o, SUB), :]
                       + cw_exst[pl.ds(u * SUB, SUB), :].astype(jnp.float32))
                out_ref[pl.ds(o, SUB), :] = fin
                cw_ag[pl.ds(o, SUB), :] = fin.astype(jnp.bfloat16)
                if DO_AG:
                    prev[u] = ag_rd(u, cw_ag, off_cw, 0, right)
                    prev[u].start()
            for u in range(2):
                ccwx[u].wait_recv()
                ccwx[u].wait_send()
                o = off_ccw + u * SUB
                fin = (out_ref[pl.ds(HALF + o, SUB), :]
                       + ccw_exst[pl.ds(u * SUB, SUB), :].astype(jnp.float32))
                out_ref[pl.ds(HALF + o, SUB), :] = fin
                ccw_ag[pl.ds(o, SUB), :] = fin.astype(jnp.bfloat16)
                if DO_AG:
                    prev[2 + u] = ag_rd(2 + u, ccw_ag, off_ccw, 0, left)
                    prev[2 + u].start()
            if DO_AG:
                pending_sends += prev

        for s in range(1, HOPS8 if DO_AG else 1):
            cw_off = chunk8(1 - s) * CHUNK8
            ccw_off = chunk8(s - 1) * CHUNK8
            for i, (buf, base, dev) in enumerate(
                    [(cw_ag, cw_off, right), (cw_ag, cw_off, right),
                     (ccw_ag, ccw_off, left), (ccw_ag, ccw_off, left)]):
                prev[i].wait_recv()
                prev[i] = ag_rd(i, buf, base, s, dev)
                prev[i].start()
            pending_sends += prev
            out_ref[pl.ds(cw_off, CHUNK8), :] = (
                cw_ag[pl.ds(cw_off, CHUNK8), :].astype(jnp.float32))
            out_ref[pl.ds(HALF + ccw_off, CHUNK8), :] = (
                ccw_ag[pl.ds(ccw_off, CHUNK8), :].astype(jnp.float32))
        if DO_AG:
            for r in prev:
                r.wait_recv()
            o = chunk8(-HOPS8 + 1) * CHUNK8
            out_ref[pl.ds(o, CHUNK8), :] = (
                cw_ag[pl.ds(o, CHUNK8), :].astype(jnp.float32))
            o = chunk8(HOPS8 - 1) * CHUNK8
            out_ref[pl.ds(HALF + o, CHUNK8), :] = (
                ccw_ag[pl.ds(o, CHUNK8), :].astype(jnp.float32))

        for r in pending_sends:
            r.wait_send()

    out = pl.pallas_call(
        body,
        out_shape=jax.ShapeDtypeStruct((ROWS, C_OUT), jnp.float32),
        in_specs=[pl.BlockSpec(memory_space=pltpu.VMEM)] * 3,
        out_specs=pl.BlockSpec(memory_space=pltpu.VMEM),
        scratch_shapes=[
            pltpu.VMEM((HOPS8, CHUNK8, C_OUT), jnp.bfloat16),
            pltpu.VMEM((HOPS8, CHUNK8, C_OUT), jnp.bfloat16),
            pltpu.VMEM((CHUNK8, C_OUT), jnp.bfloat16),
            pltpu.VMEM((CHUNK8, C_OUT), jnp.bfloat16),
            pltpu.VMEM((HALF, C_OUT), jnp.bfloat16),
            pltpu.VMEM((HALF, C_OUT), jnp.bfloat16),
        ] + [pltpu.SemaphoreType.DMA((HOPS8,))] * 16
          + [pltpu.SemaphoreType.DMA((4,))] * 2,
        compiler_params=pltpu.CompilerParams(collective_id=0),
    )(x, k, Wp)
    return out.reshape(B, S, C_OUT)
